# baseline (speedup 1.0000x reference)
import functools

import jax
import jax.numpy as jnp
from jax import lax
from jax.experimental import pallas as pl
from jax.experimental.pallas import tpu as pltpu
from jax.experimental.pallas import tpu_sc as plsc

NC = 2
NS = 16
NW = NC * NS
LANES = 16


def _sc_gather_mean(ctx_rs, emb, batch, ctx_len, dim):
    rows_per_chunk = 4
    idx_per_chunk = rows_per_chunk * ctx_len
    rows_per_worker = batch // NW
    chunks_per_worker = rows_per_worker // rows_per_chunk
    n_k = dim // LANES
    scale = 1.0 / ctx_len

    mesh = plsc.VectorSubcoreMesh(
        core_axis_name="c", subcore_axis_name="s",
        num_cores=NC, num_subcores=NS)

    @functools.partial(
        pl.kernel,
        out_type=jax.ShapeDtypeStruct((batch, dim), jnp.float32),
        mesh=mesh,
        scratch_types=[
            pltpu.VMEM((chunks_per_worker, idx_per_chunk), jnp.int32),
            pltpu.VMEM((idx_per_chunk, dim), jnp.float32),
            pltpu.VMEM((rows_per_worker, dim), jnp.float32),
            pltpu.SemaphoreType.DMA,
        ],
        compiler_params=pltpu.CompilerParams(use_tc_tiling_on_sc=False),
    )
    def sc_kernel(ctx_hbm, emb_hbm, out_hbm, idx_v, buf_v, acc_v, sem):
        wid = lax.axis_index("s") * NC + lax.axis_index("c")
        pltpu.sync_copy(ctx_hbm.at[pl.ds(wid * chunks_per_worker,
                                         chunks_per_worker)], idx_v)

        def chunk_body(c, _):
            pltpu.async_copy(emb_hbm.at[idx_v.at[c]], buf_v, sem).wait()
            for rr in range(rows_per_chunk):
                for kk in range(n_k):
                    sl = pl.ds(kk * LANES, LANES)
                    v = buf_v[rr * ctx_len, sl]
                    for w in range(1, ctx_len):
                        v = v + buf_v[rr * ctx_len + w, sl]
                    acc_v[c * rows_per_chunk + rr, sl] = v * scale
            return 0

        lax.fori_loop(0, chunks_per_worker, chunk_body, 0)
        pltpu.sync_copy(
            acc_v, out_hbm.at[pl.ds(wid * rows_per_worker, rows_per_worker)])

    return sc_kernel(ctx_rs, emb)


def _tc_matmul(mean, w, b2, batch, dim, vocab):
    vt = 512
    grid = (vocab + vt - 1) // vt

    def body(a_ref, w_ref, b_ref, o_ref):
        o_ref[...] = lax.dot_general(
            a_ref[...], w_ref[...], (((1,), (1,)), ((), ())),
            preferred_element_type=jnp.float32,
            precision=lax.Precision.HIGHEST) + b_ref[...]

    return pl.pallas_call(
        body,
        grid=(grid,),
        in_specs=[
            pl.BlockSpec((batch, dim), lambda v: (0, 0)),
            pl.BlockSpec((vt, dim), lambda v: (v, 0)),
            pl.BlockSpec((1, vt), lambda v: (0, v)),
        ],
        out_specs=pl.BlockSpec((batch, vt), lambda v: (0, v)),
        out_shape=jax.ShapeDtypeStruct((batch, vocab), jnp.float32),
        compiler_params=pltpu.CompilerParams(
            dimension_semantics=("arbitrary",)),
    )(mean, w, b2)


def kernel(context_indices, embeddings, linear_w, linear_b):
    batch, ctx_len = context_indices.shape
    vocab, dim = embeddings.shape
    rows_per_chunk = 4
    ctx_rs = context_indices.astype(jnp.int32).reshape(
        batch // rows_per_chunk, rows_per_chunk * ctx_len)
    mean = _sc_gather_mean(ctx_rs, embeddings, batch, ctx_len, dim)
    b2 = linear_b.reshape(1, vocab)
    return _tc_matmul(mean, linear_w, b2, batch, dim, vocab)
